# Initial kernel scaffold; baseline (speedup 1.0000x reference)
import jax
import jax.numpy as jnp
from jax.experimental import pallas as pl


def kernel(query, memory_bank, fcn_w, fcn_b):
    raise NotImplementedError("write your pallas kernel here")


# speedup vs baseline: 7.7774x; 7.7774x over previous
import jax
import jax.numpy as jnp
from jax import lax
from jax.experimental import pallas as pl
from jax.experimental.pallas import tpu as pltpu

_N = 4096
_D = 96
_M = 16384
_QB = 128
_C = 2048
_INF = 3.0e38


def _knn_body(q_ref, mb_ref, wq_ref, wn_ref, t_ref, s_ref, m2_ref):
    q = q_ref[...]
    nc = _M // _C

    @pl.when(pl.program_id(0) == 0)
    def _build_m2():
        for c in range(nc):
            mb = mb_ref[pl.ds(c * _C, _C), :]
            m2_ref[pl.ds(c * _C, _C), :] = jnp.sum(mb * mb, axis=1,
                                                   keepdims=True)

    for c in range(nc):
        mb = mb_ref[pl.ds(c * _C, _C), :]
        qm = lax.dot_general(mb, q, (((1,), (1,)), ((), ())),
                             preferred_element_type=jnp.float32)
        s_ref[pl.ds(c * _C, _C), :] = m2_ref[pl.ds(c * _C, _C), :] - 2.0 * qm
    t = None
    for _ in range(4):
        cmins = []
        for c in range(nc):
            s = s_ref[pl.ds(c * _C, _C), :]
            if t is not None:
                s = jnp.where(s <= t, _INF, s)
            cmins.append(jnp.min(s, axis=0, keepdims=True))
        t = jnp.min(jnp.concatenate(cmins, axis=0), axis=0, keepdims=True)
    acc = jnp.zeros((_QB, _D), jnp.float32)
    for c in range(nc):
        s = s_ref[pl.ds(c * _C, _C), :]
        sel = (s <= t).astype(jnp.float32)
        mb = mb_ref[pl.ds(c * _C, _C), :]
        acc = acc + lax.dot_general(sel, mb, (((0,), (0,)), ((), ())),
                                    preferred_element_type=jnp.float32)
    nmean = acc * 0.25
    t_ref[...] = (
        lax.dot_general(q, wq_ref[...], (((1,), (0,)), ((), ())),
                        preferred_element_type=jnp.float32)
        + lax.dot_general(nmean, wn_ref[...], (((1,), (0,)), ((), ())),
                          preferred_element_type=jnp.float32))


def _shift_body(t_ref, b_ref, out_ref):
    zrow = jnp.zeros((1, 64), jnp.float32)
    zcol = jnp.zeros((64, 1), jnp.float32)
    acc = jnp.zeros((64, 64), jnp.float32)
    for ky in range(3):
        for kx in range(3):
            p = t_ref[ky * 3 + kx]
            if ky == 0:
                p = jnp.concatenate([zrow, p[:63, :]], axis=0)
            elif ky == 2:
                p = jnp.concatenate([p[1:, :], zrow], axis=0)
            if kx == 0:
                p = jnp.concatenate([zcol, p[:, :63]], axis=1)
            elif kx == 2:
                p = jnp.concatenate([p[:, 1:], zcol], axis=1)
            acc = acc + p
    out_ref[...] = acc + b_ref[0, 0]


def kernel(query, memory_bank, fcn_w, fcn_b):
    w = fcn_w[0].reshape(2 * _D, 9)
    wq, wn = w[:_D], w[_D:]
    t9 = pl.pallas_call(
        _knn_body,
        grid=(_N // _QB,),
        in_specs=[
            pl.BlockSpec((_QB, _D), lambda i: (i, 0)),
            pl.BlockSpec((_M, _D), lambda i: (0, 0)),
            pl.BlockSpec((_D, 9), lambda i: (0, 0)),
            pl.BlockSpec((_D, 9), lambda i: (0, 0)),
        ],
        out_specs=pl.BlockSpec((_QB, 9), lambda i: (i, 0)),
        out_shape=jax.ShapeDtypeStruct((_N, 9), jnp.float32),
        scratch_shapes=[pltpu.VMEM((_M, _QB), jnp.float32),
                        pltpu.VMEM((_M, 1), jnp.float32)],
        compiler_params=pltpu.CompilerParams(
            dimension_semantics=("arbitrary",)),
    )(query, memory_bank, wq, wn)

    t3 = t9.T.reshape(9, 64, 64)
    out = pl.pallas_call(
        _shift_body,
        in_specs=[
            pl.BlockSpec((9, 64, 64), lambda: (0, 0, 0)),
            pl.BlockSpec((1, 1), lambda: (0, 0), memory_space=pltpu.SMEM),
        ],
        out_specs=pl.BlockSpec((64, 64), lambda: (0, 0)),
        out_shape=jax.ShapeDtypeStruct((64, 64), jnp.float32),
    )(t3, fcn_b.reshape(1, 1))
    return out.reshape(1, 1, 64, 64)
